# baseline (speedup 1.0000x reference)
import functools

import jax
import jax.numpy as jnp
from jax import lax
from jax.experimental import pallas as pl
from jax.experimental.pallas import tpu as pltpu
from jax.experimental.pallas import tpu_sc as plsc

_NC = 2
_NS = 16
_NW = _NC * _NS

_TRANS_C = 1000

_BC = 16
_SC = 100

_MESH = plsc.VectorSubcoreMesh(core_axis_name="c", subcore_axis_name="s")
_PARAMS = pltpu.CompilerParams(
    use_tc_tiling_on_sc=False, needs_layout_passes=False)


def _make_transpose_call(V, D):
    n_blocks = V // _TRANS_C
    lanes = 16

    @functools.partial(
        pl.kernel,
        mesh=_MESH,
        compiler_params=_PARAMS,
        out_type=jax.ShapeDtypeStruct((V, D), jnp.float32),
        scratch_types=[
            pltpu.VMEM((D, _TRANS_C), jnp.float32),
            pltpu.VMEM((_TRANS_C, D), jnp.float32),
            pltpu.SemaphoreType.DMA,
        ],
    )
    def trans_call(tab_t_hbm, tab_rm_hbm, in_v, out_v, sem):
        wid = lax.axis_index("s") * _NC + lax.axis_index("c")
        n_mine = lax.div(n_blocks - wid - 1 + _NW, _NW)

        def block_body(i, carry):
            blk = wid + i * _NW
            r0 = blk * _TRANS_C
            pltpu.async_copy(
                tab_t_hbm.at[:, pl.ds(r0, _TRANS_C)], in_v, sem).wait()

            def row_body(r, carry2):
                rr = jnp.full((lanes,), r, dtype=jnp.int32)
                for half in range(D // lanes):
                    dd = lax.iota(jnp.int32, lanes) + half * lanes
                    out_v[r, pl.ds(half * lanes, lanes)] = (
                        plsc.load_gather(in_v, [dd, rr]))
                return carry2
            lax.fori_loop(0, _TRANS_C, row_body, 0)

            pltpu.async_copy(
                out_v, tab_rm_hbm.at[pl.ds(r0, _TRANS_C)], sem).wait()
            return carry

        lax.fori_loop(0, n_mine, block_body, 0)

    return trans_call


def _make_gather_call(B, S, V, D):
    b_per_w = B // _NW
    nb = b_per_w // _BC
    ns = S // _SC
    lanes = 16

    @functools.partial(
        pl.kernel,
        mesh=_MESH,
        compiler_params=_PARAMS,
        out_type=jax.ShapeDtypeStruct((S, D, B), jnp.float32),
        scratch_types=[
            pltpu.VMEM((_BC * ns, _SC), jnp.int32),
            pltpu.VMEM((_BC * _SC, D), jnp.float32),
            pltpu.VMEM((_SC, D, _BC), jnp.float32),
            pltpu.VMEM((S, D), jnp.float32),
            pltpu.SemaphoreType.DMA,
            pltpu.SemaphoreType.DMA,
        ],
    )
    def gather_call(x2_hbm, tab_rm_hbm, pos_hbm, out_hbm,
                    idx_v, rows_v, trans_v, pos_v, gsem, msem):
        wid = lax.axis_index("s") * _NC + lax.axis_index("c")
        b_base = wid * b_per_w

        pltpu.async_copy(pos_hbm, pos_v, msem).wait()

        def group_body(g, carry):
            b0 = pl.multiple_of(b_base + g * _BC, _BC)

            pltpu.async_copy(
                x2_hbm.at[pl.ds(b0 * ns, _BC * ns)], idx_v, msem
            ).wait()

            for h in range(ns):
                s0 = h * _SC
                descs = []
                for j in range(_BC):
                    descs.append(pltpu.async_copy(
                        tab_rm_hbm.at[idx_v.at[j * ns + h]],
                        rows_v.at[pl.ds(j * _SC, _SC)],
                        gsem,
                    ))
                for dsc in descs:
                    dsc.wait()

                def pos_body(s, carry2):
                    bb = lax.iota(jnp.int32, lanes) * _SC + s
                    pos_halves = [pos_v[s0 + s, pl.ds(k * lanes, lanes)]
                                  for k in range(D // lanes)]
                    for d in range(D):
                        dd = jnp.full((lanes,), d, dtype=jnp.int32)
                        v = plsc.load_gather(rows_v, [bb, dd])
                        trans_v[s, d, :] = (
                            v + pos_halves[d // lanes][d % lanes])
                    return carry2
                lax.fori_loop(0, _SC, pos_body, 0)

                pltpu.async_copy(
                    trans_v,
                    out_hbm.at[pl.ds(s0, _SC), :, pl.ds(b0, _BC)],
                    msem,
                ).wait()
            return carry

        lax.fori_loop(0, nb, group_body, 0)

    return gather_call


def kernel(x, table, pos_encoding):
    B, S = x.shape
    V, D = table.shape
    tab_t = table.T
    pos2d = pos_encoding[0, :S, :]
    x2d = x.reshape((B * S) // _SC, _SC)
    tab_rm = _make_transpose_call(V, D)(tab_t)
    out_phys = _make_gather_call(B, S, V, D)(x2d, tab_rm, pos2d)
    return out_phys.transpose(2, 0, 1)


# speedup vs baseline: 2.7695x; 2.7695x over previous
import functools

import jax
import jax.numpy as jnp
from jax import lax
from jax.experimental import pallas as pl
from jax.experimental.pallas import tpu as pltpu
from jax.experimental.pallas import tpu_sc as plsc

_NC = 2
_NS = 16
_NW = _NC * _NS

_TRANS_C = 1000

_BC = 16
_SC = 100

_MESH = plsc.VectorSubcoreMesh(core_axis_name="c", subcore_axis_name="s")
_PARAMS = pltpu.CompilerParams(
    use_tc_tiling_on_sc=False, needs_layout_passes=False)


def _make_transpose_call(V, D):
    n_blocks = V // _TRANS_C
    lanes = 16

    @functools.partial(
        pl.kernel,
        mesh=_MESH,
        compiler_params=_PARAMS,
        out_type=jax.ShapeDtypeStruct((V, D), jnp.float32),
        scratch_types=[
            pltpu.VMEM((D, _TRANS_C), jnp.float32),
            pltpu.VMEM((_TRANS_C, D), jnp.float32),
            pltpu.SemaphoreType.DMA,
        ],
    )
    def trans_call(tab_t_hbm, tab_rm_hbm, in_v, out_v, sem):
        wid = lax.axis_index("s") * _NC + lax.axis_index("c")
        n_mine = lax.div(n_blocks - wid - 1 + _NW, _NW)

        def block_body(i, carry):
            blk = wid + i * _NW
            r0 = blk * _TRANS_C
            pltpu.async_copy(
                tab_t_hbm.at[:, pl.ds(r0, _TRANS_C)], in_v, sem).wait()

            def row_body(r, carry2):
                rr = jnp.full((lanes,), r, dtype=jnp.int32)
                for half in range(D // lanes):
                    dd = lax.iota(jnp.int32, lanes) + half * lanes
                    out_v[r, pl.ds(half * lanes, lanes)] = (
                        plsc.load_gather(in_v, [dd, rr]))
                return carry2
            lax.fori_loop(0, _TRANS_C, row_body, 0)

            pltpu.async_copy(
                out_v, tab_rm_hbm.at[pl.ds(r0, _TRANS_C)], sem).wait()
            return carry

        lax.fori_loop(0, n_mine, block_body, 0)

    return trans_call


def _make_gather_call(B, S, V, D):
    b_per_w = B // _NW
    nb = b_per_w // _BC
    ns = S // _SC
    lanes = 16

    @functools.partial(
        pl.kernel,
        mesh=_MESH,
        compiler_params=_PARAMS,
        out_type=jax.ShapeDtypeStruct((S, D, B), jnp.float32),
        scratch_types=[
            pltpu.VMEM((_BC * ns, _SC), jnp.int32),
            pltpu.VMEM((_BC * _SC, D), jnp.float32),
            pltpu.VMEM((_SC, D, _BC), jnp.float32),
            pltpu.VMEM((S, D), jnp.float32),
            pltpu.SemaphoreType.DMA,
            pltpu.SemaphoreType.DMA,
        ],
    )
    def gather_call(x2_hbm, tab_rm_hbm, pos_hbm, out_hbm,
                    idx_v, rows_v, trans_v, pos_v, gsem, msem):
        wid = lax.axis_index("s") * _NC + lax.axis_index("c")
        b_base = wid * b_per_w

        pltpu.async_copy(pos_hbm, pos_v, msem).wait()

        def group_body(g, carry):
            b0 = pl.multiple_of(b_base + g * _BC, _BC)

            pltpu.async_copy(
                x2_hbm.at[pl.ds(b0 * ns, _BC * ns)], idx_v, msem
            ).wait()

            for h in range(ns):
                s0 = h * _SC
                descs = []
                for j in range(_BC):
                    descs.append(pltpu.async_copy(
                        tab_rm_hbm.at[idx_v.at[j * ns + h]],
                        rows_v.at[pl.ds(j * _SC, _SC)],
                        gsem,
                    ))
                for dsc in descs:
                    dsc.wait()

                def pos_body(s, carry2):
                    bb = lax.iota(jnp.int32, lanes) * _SC + s
                    pos_halves = [pos_v[s0 + s, pl.ds(k * lanes, lanes)]
                                  for k in range(D // lanes)]
                    for d in range(D):
                        dd = jnp.full((lanes,), d, dtype=jnp.int32)
                        v = plsc.load_gather(rows_v, [bb, dd])
                        trans_v[s, d, :] = (
                            v + pos_halves[d // lanes][d % lanes])
                    return carry2
                lax.fori_loop(0, _SC, pos_body, 0)

                pltpu.async_copy(
                    trans_v,
                    out_hbm.at[pl.ds(s0, _SC), :, pl.ds(b0, _BC)],
                    msem,
                ).wait()
            return carry

        lax.fori_loop(0, nb, group_body, 0)

    return gather_call


def kernel(x, table, pos_encoding):
    B, S = x.shape
    V, D = table.shape
    pos2d = pos_encoding[0, :S, :]
    x2d = x.reshape((B * S) // _SC, _SC)
    out_phys = _make_gather_call(B, S, V, D)(x2d, table, pos2d)
    return out_phys.transpose(2, 0, 1)


# speedup vs baseline: 3.0891x; 1.1154x over previous
import functools

import jax
import jax.numpy as jnp
from jax import lax
from jax.experimental import pallas as pl
from jax.experimental.pallas import tpu as pltpu
from jax.experimental.pallas import tpu_sc as plsc

_NC = 2
_NS = 16
_NW = _NC * _NS

_BC = 32
_SCK = 40
_LN = 16

_MESH = plsc.VectorSubcoreMesh(core_axis_name="c", subcore_axis_name="s")
_PARAMS = pltpu.CompilerParams(
    use_tc_tiling_on_sc=False, needs_layout_passes=False)


def _make_gather_call(B, S, V, D):
    b_per_w = B // _NW
    nb = b_per_w // _BC
    ns = S // _SCK
    dh_n = D // 8

    @functools.partial(
        pl.kernel,
        mesh=_MESH,
        compiler_params=_PARAMS,
        out_type=jax.ShapeDtypeStruct((S, dh_n, _NW, 8, 128), jnp.float32),
        scratch_types=[
            pltpu.VMEM((_BC, _SCK), jnp.int32),
            pltpu.VMEM((_BC * _SCK, D), jnp.float32),
            pltpu.VMEM((_SCK, dh_n, 8, _BC), jnp.float32),
            pltpu.VMEM((S, D), jnp.float32),
            pltpu.SemaphoreType.DMA,
            pltpu.SemaphoreType.DMA,
        ],
    )
    def gather_call(x_hbm, tab_hbm, pos_hbm, out_hbm,
                    idx_v, rows_v, trans_v, pos_v, gsem, msem):
        wid = lax.axis_index("s") * _NC + lax.axis_index("c")
        b_base = wid * b_per_w

        pltpu.async_copy(pos_hbm, pos_v, msem).wait()

        def chunk_body(g, carry):
            bl0 = pl.multiple_of((g // ns) * _BC, _BC)
            s0 = pl.multiple_of((g % ns) * _SCK, _SCK)

            pltpu.async_copy(
                x_hbm.at[pl.ds(b_base + bl0, _BC), pl.ds(s0, _SCK)],
                idx_v, msem,
            ).wait()

            descs = []
            for j in range(_BC):
                descs.append(pltpu.async_copy(
                    tab_hbm.at[idx_v.at[j]],
                    rows_v.at[pl.ds(j * _SCK, _SCK)],
                    gsem,
                ))
            for dsc in descs:
                dsc.wait()

            def add_body(s, carry2):
                for h in range(D // _LN):
                    p = pos_v[s0 + s, pl.ds(h * _LN, _LN)]
                    for j in range(_BC):
                        r = j * _SCK + s
                        rows_v[r, pl.ds(h * _LN, _LN)] = (
                            rows_v[r, pl.ds(h * _LN, _LN)] + p)
                return carry2
            lax.fori_loop(0, _SCK, add_body, 0)

            jj_base = lax.iota(jnp.int32, _LN) * _SCK
            def tr_body(s, carry2):
                row0 = jj_base + s
                for d in range(D):
                    dd = jnp.full((_LN,), d, dtype=jnp.int32)
                    for q in range(_BC // _LN):
                        v = plsc.load_gather(
                            rows_v, [row0 + q * (_LN * _SCK), dd])
                        trans_v[s, d // 8, d % 8, pl.ds(q * _LN, _LN)] = v
                return carry2
            lax.fori_loop(0, _SCK, tr_body, 0)

            for dh in range(dh_n):
                pltpu.async_copy(
                    trans_v.at[:, dh],
                    out_hbm.at[pl.ds(s0, _SCK), dh, wid, :,
                               pl.ds(bl0, _BC)],
                    msem,
                ).wait()
            return carry

        lax.fori_loop(0, nb * ns, chunk_body, 0)

    return gather_call


def kernel(x, table, pos_encoding):
    B, S = x.shape
    V, D = table.shape
    pos2d = pos_encoding[0, :S, :]
    out5 = _make_gather_call(B, S, V, D)(x, table, pos2d)
    out = out5.transpose(2, 4, 0, 1, 3).reshape(B, S, D)
    return out


# speedup vs baseline: 4.5728x; 1.4803x over previous
import functools

import jax
import jax.numpy as jnp
from jax import lax
from jax.experimental import pallas as pl
from jax.experimental.pallas import tpu as pltpu
from jax.experimental.pallas import tpu_sc as plsc

_NC = 2
_NS = 16
_NW = _NC * _NS

_BC = 32
_SCK = 40
_LN = 16

_MESH = plsc.VectorSubcoreMesh(core_axis_name="c", subcore_axis_name="s")
_PARAMS = pltpu.CompilerParams(
    use_tc_tiling_on_sc=False, needs_layout_passes=False)


def _make_gather_call(B, S, V, D):
    b_per_w = B // _NW
    nb = b_per_w // _BC
    ns = S // _SCK
    dh_n = D // 8

    @functools.partial(
        pl.kernel,
        mesh=_MESH,
        compiler_params=_PARAMS,
        out_type=jax.ShapeDtypeStruct((S, dh_n, _NW, 8, 128), jnp.float32),
        scratch_types=[
            pltpu.VMEM((_BC, _SCK), jnp.int32),
            pltpu.VMEM((_BC * _SCK, D), jnp.float32),
            pltpu.VMEM((_SCK, dh_n, 8, _BC + 1), jnp.float32),
            pltpu.VMEM((S, D), jnp.float32),
            pltpu.SemaphoreType.DMA,
            pltpu.SemaphoreType.DMA,
        ],
    )
    def gather_call(x_hbm, tab_hbm, pos_hbm, out_hbm,
                    idx_v, rows_v, trans_v, pos_v, gsem, msem):
        wid = lax.axis_index("s") * _NC + lax.axis_index("c")
        b_base = wid * b_per_w

        pltpu.async_copy(pos_hbm, pos_v, msem).wait()

        def chunk_body(g, carry):
            bl0 = pl.multiple_of((g // ns) * _BC, _BC)
            s0 = pl.multiple_of((g % ns) * _SCK, _SCK)

            pltpu.async_copy(
                x_hbm.at[pl.ds(b_base + bl0, _BC), pl.ds(s0, _SCK)],
                idx_v, msem,
            ).wait()

            descs = []
            for j in range(_BC):
                descs.append(pltpu.async_copy(
                    tab_hbm.at[idx_v.at[j]],
                    rows_v.at[pl.ds(j * _SCK, _SCK)],
                    gsem,
                ))
            for dsc in descs:
                dsc.wait()

            dd = lax.iota(jnp.int32, _LN)
            dh_c = [(dd + h * _LN) // 8 for h in range(D // _LN)]
            dl_c = [(dd + h * _LN) % 8 for h in range(D // _LN)]
            def tr_body(s, carry2):
                s_vec = jnp.full((_LN,), 0, jnp.int32) + s
                pos_h = [pos_v[s0 + s, pl.ds(h * _LN, _LN)]
                         for h in range(D // _LN)]
                for j in range(_BC):
                    r = j * _SCK + s
                    j_vec = jnp.full((_LN,), j, dtype=jnp.int32)
                    for h in range(D // _LN):
                        v = rows_v[r, pl.ds(h * _LN, _LN)] + pos_h[h]
                        plsc.store_scatter(
                            trans_v, [s_vec, dh_c[h], dl_c[h], j_vec], v)
                return carry2
            lax.fori_loop(0, _SCK, tr_body, 0)

            for dh in range(dh_n):
                pltpu.async_copy(
                    trans_v.at[:, dh, :, pl.ds(0, _BC)],
                    out_hbm.at[pl.ds(s0, _SCK), dh, wid, :,
                               pl.ds(bl0, _BC)],
                    msem,
                ).wait()
            return carry

        lax.fori_loop(0, nb * ns, chunk_body, 0)

    return gather_call


def kernel(x, table, pos_encoding):
    B, S = x.shape
    V, D = table.shape
    pos2d = pos_encoding[0, :S, :]
    out5 = _make_gather_call(B, S, V, D)(x, table, pos2d)
    out = out5.transpose(2, 4, 0, 1, 3).reshape(B, S, D)
    return out
